# baseline (speedup 1.0000x reference)
import jax
import jax.numpy as jnp
from jax import lax
from jax.experimental import pallas as pl
from jax.experimental.pallas import tpu as pltpu
from jax.experimental.pallas import tpu_sc as plsc

_NC = 2
_NS = 16
_NW = _NC * _NS
_CHUNK = 625


def _copy_body(u_hbm, i_hbm, ou_hbm, oi_hbm, bufs, rsem, wsem):
    wid = lax.axis_index("s") * _NC + lax.axis_index("c")

    tasks = []
    for (src, dst) in ((u_hbm, ou_hbm), (i_hbm, oi_hbm)):
        rpw = src.shape[0] // _NW
        base = wid * rpw
        for k in range(rpw // _CHUNK):
            tasks.append((src, dst, base + k * _CHUNK))

    reads, writes = [], []
    for k, (src, dst, off) in enumerate(tasks):
        b = k % 2
        reads.append(pltpu.make_async_copy(
            src.at[pl.ds(off, _CHUNK), :], bufs.at[b], rsem.at[b]))
        writes.append(pltpu.make_async_copy(
            bufs.at[b], dst.at[pl.ds(off, _CHUNK), :], wsem.at[b]))
    n = len(tasks)
    reads[0].start()
    for k in range(n):
        reads[k].wait()
        if k + 1 < n:
            if k >= 1:
                writes[k - 1].wait()
            reads[k + 1].start()
        writes[k].start()
    writes[n - 1].wait()
    if n >= 2:
        writes[n - 2].wait()


def kernel(user_emb, item_emb):
    n_u, d = user_emb.shape
    n_i, _ = item_emb.shape
    mesh = plsc.VectorSubcoreMesh(core_axis_name="c", subcore_axis_name="s",
                                  num_cores=_NC, num_subcores=_NS)
    run = pl.kernel(
        _copy_body,
        out_type=[
            jax.ShapeDtypeStruct((n_u, d), user_emb.dtype),
            jax.ShapeDtypeStruct((n_i, d), item_emb.dtype),
        ],
        mesh=mesh,
        scratch_types=[
            pltpu.VMEM((2, _CHUNK, 64), jnp.float32),
            pltpu.SemaphoreType.DMA((2,)),
            pltpu.SemaphoreType.DMA((2,)),
        ],
        compiler_params=pltpu.CompilerParams(use_tc_tiling_on_sc=False),
    )
    out_u, out_i = run(user_emb, item_emb)
    return (out_u, out_i)


# speedup vs baseline: 8.0486x; 8.0486x over previous
import jax
import jax.numpy as jnp
from jax.experimental import pallas as pl
from jax.experimental.pallas import tpu as pltpu

_BLOCK_COLS = 16384


def _copy_body(u_ref, i_ref, ou_ref, oi_ref):
    ou_ref[...] = u_ref[...]
    oi_ref[...] = i_ref[...]


def kernel(user_emb, item_emb):
    n_u, d = user_emb.shape
    n_i, _ = item_emb.shape
    ut = user_emb.T
    it = item_emb.T
    grid = (pl.cdiv(n_u, _BLOCK_COLS),)
    out_u, out_i = pl.pallas_call(
        _copy_body,
        grid=grid,
        in_specs=[
            pl.BlockSpec((d, _BLOCK_COLS), lambda c: (0, c)),
            pl.BlockSpec((d, _BLOCK_COLS), lambda c: (0, c)),
        ],
        out_specs=[
            pl.BlockSpec((d, _BLOCK_COLS), lambda c: (0, c)),
            pl.BlockSpec((d, _BLOCK_COLS), lambda c: (0, c)),
        ],
        out_shape=[
            jax.ShapeDtypeStruct((d, n_u), user_emb.dtype),
            jax.ShapeDtypeStruct((d, n_i), item_emb.dtype),
        ],
        compiler_params=pltpu.CompilerParams(
            dimension_semantics=("parallel",),
        ),
    )(ut, it)
    return (out_u.T, out_i.T)


# speedup vs baseline: 8.0622x; 1.0017x over previous
import jax
import jax.numpy as jnp
from jax.experimental import pallas as pl
from jax.experimental.pallas import tpu as pltpu

_BLOCK_COLS = 24576


def _copy_body(u_ref, i_ref, ou_ref, oi_ref):
    ou_ref[...] = u_ref[...]
    oi_ref[...] = i_ref[...]


def kernel(user_emb, item_emb):
    n_u, d = user_emb.shape
    n_i, _ = item_emb.shape
    ut = user_emb.T
    it = item_emb.T
    grid = (pl.cdiv(n_u, _BLOCK_COLS),)
    out_u, out_i = pl.pallas_call(
        _copy_body,
        grid=grid,
        in_specs=[
            pl.BlockSpec((d, _BLOCK_COLS), lambda c: (0, c)),
            pl.BlockSpec((d, _BLOCK_COLS), lambda c: (0, c)),
        ],
        out_specs=[
            pl.BlockSpec((d, _BLOCK_COLS), lambda c: (0, c)),
            pl.BlockSpec((d, _BLOCK_COLS), lambda c: (0, c)),
        ],
        out_shape=[
            jax.ShapeDtypeStruct((d, n_u), user_emb.dtype),
            jax.ShapeDtypeStruct((d, n_i), item_emb.dtype),
        ],
        compiler_params=pltpu.CompilerParams(
            dimension_semantics=("parallel",),
        ),
    )(ut, it)
    return (out_u.T, out_i.T)
